# baseline (speedup 1.0000x reference)
import functools

import jax
import jax.numpy as jnp
from jax import lax
from jax.experimental import pallas as pl
from jax.experimental.pallas import tpu as pltpu
from jax.experimental.pallas import tpu_sc as plsc

_NC = 2
_NS = 16
_CH = 128


def _sc_aggregate(feature, edge_index, zrow):
    N, D = feature.shape
    E = edge_index.shape[1]
    NW = _NC * _NS
    rows_per_sub = ((N + _CH * _NS - 1) // (_CH * _NS)) * _CH
    NPAD = rows_per_sub * _NS

    G = -(-E // (_CH * NW))
    G = ((G + 3) // 4) * 4
    E_pad = G * NW * _CH
    if E_pad > E:
        pad = jnp.concatenate(
            [jnp.zeros((1, E_pad - E), jnp.int32),
             jnp.full((1, E_pad - E), N, jnp.int32)], axis=0)
        edge_index = jnp.concatenate([edge_index, pad], axis=1)
    edge2 = edge_index
    mesh = plsc.VectorSubcoreMesh(core_axis_name="c", subcore_axis_name="s")

    @functools.partial(
        pl.kernel,
        out_type=jax.ShapeDtypeStruct((_NC, NPAD, D), jnp.float32),
        mesh=mesh,
        scratch_types=[
            pltpu.VMEM((2, 2, _CH), jnp.int32),
            pltpu.VMEM((_CH, D), jnp.float32),
            pltpu.VMEM((_CH, D), jnp.float32),
            pltpu.VMEM_SHARED((NPAD, D), jnp.float32),
            [pltpu.SemaphoreType.DMA] * 2,
        ],
    )
    def agg_kernel(feat_hbm, edge_hbm, zrow_hbm, out_hbm,
                   idx, rows0, rows1, acc, gsems):
        c = lax.axis_index("c")
        s = lax.axis_index("s")
        w = c * _NS + s
        c0 = w * G
        rows = (rows0, rows1)

        def idx_load(k, slot):
            off = (c0 + k) * _CH
            pltpu.sync_copy(edge_hbm.at[0, pl.ds(off, _CH)], idx.at[slot, 0])
            pltpu.sync_copy(edge_hbm.at[1, pl.ds(off, _CH)], idx.at[slot, 1])

        pltpu.sync_copy(zrow_hbm, rows0)

        @pl.loop(0, rows_per_sub // _CH)
        def _(j):
            pltpu.sync_copy(
                rows0, acc.at[pl.ds(s * rows_per_sub + j * _CH, _CH), :])

        plsc.subcore_barrier()

        idx_load(0, 0)
        pltpu.async_copy(feat_hbm.at[idx.at[0, 0]], rows0, gsems[0])

        @pl.loop(0, G - 2, step=2)
        def _(j):
            for b in range(2):
                k = j + b
                nb = 1 - b
                idx_load(k + 1, nb)
                pltpu.async_copy(feat_hbm.at[idx.at[nb, 0]], rows[nb],
                                 gsems[nb])
                pltpu.make_async_copy(feat_hbm.at[idx.at[b, 0]], rows[b],
                                      gsems[b]).wait()
                pltpu.sync_copy(rows[b], acc.at[idx.at[b, 1]], add=True)

        idx_load(G - 1, 1)
        pltpu.async_copy(feat_hbm.at[idx.at[1, 0]], rows1, gsems[1])
        pltpu.make_async_copy(feat_hbm.at[idx.at[0, 0]], rows0,
                              gsems[0]).wait()
        pltpu.sync_copy(rows0, acc.at[idx.at[0, 1]], add=True)
        pltpu.make_async_copy(feat_hbm.at[idx.at[1, 0]], rows1,
                              gsems[1]).wait()
        pltpu.sync_copy(rows1, acc.at[idx.at[1, 1]], add=True)

        plsc.subcore_barrier()

        pltpu.sync_copy(
            acc.at[pl.ds(s * rows_per_sub, rows_per_sub), :],
            out_hbm.at[c, pl.ds(s * rows_per_sub, rows_per_sub), :])

    return agg_kernel(feature, edge2, zrow)


def _tc_fused(feature, partials, W1t, b1, W2t, b2, gamma, beta):
    N, D = feature.shape

    def body(f_ref, p_ref, w1_ref, b1_ref, w2_ref, b2_ref, g_ref, be_ref,
             o_ref):
        x = f_ref[...] + p_ref[0, pl.ds(0, N), :] + p_ref[1, pl.ds(0, N), :]
        h = jnp.dot(x, w1_ref[...], preferred_element_type=jnp.float32,
                    precision=lax.Precision.HIGHEST) + b1_ref[...]
        h = jnp.maximum(h, 0.0)
        h = jnp.dot(h, w2_ref[...], preferred_element_type=jnp.float32,
                    precision=lax.Precision.HIGHEST) + b2_ref[...]
        h = jnp.maximum(h, 0.0)
        mean = jnp.mean(h, axis=0, keepdims=True)
        var = jnp.mean(h * h, axis=0, keepdims=True) - mean * mean
        inv = lax.rsqrt(var + 1e-5)
        o_ref[...] = (h - mean) * inv * g_ref[...] + be_ref[...]

    return pl.pallas_call(
        body,
        out_shape=jax.ShapeDtypeStruct((N, D), jnp.float32),
    )(feature, partials, W1t, b1, W2t, b2, gamma, beta)


def kernel(feature, edge_index, W1, b1, W2, b2, gamma, beta):
    D = feature.shape[1]
    zrow = jnp.zeros((_CH, D), jnp.float32)
    partials = _sc_aggregate(feature, edge_index, zrow)
    return _tc_fused(feature, partials, W1.T, b1.reshape(1, D), W2.T,
                     b2.reshape(1, D), gamma.reshape(1, D),
                     beta.reshape(1, D))


# speedup vs baseline: 2.6432x; 2.6432x over previous
import functools

import jax
import jax.numpy as jnp
from jax import lax
from jax.experimental import pallas as pl
from jax.experimental.pallas import tpu as pltpu
from jax.experimental.pallas import tpu_sc as plsc

_NC = 2
_NS = 16
_CH = 128


def _sc_aggregate(feature, edge_index, zrow):
    N, D = feature.shape
    E = edge_index.shape[1]
    NW = _NC * _NS
    rows_per_sub = ((N + _CH * _NS - 1) // (_CH * _NS)) * _CH
    NPAD = rows_per_sub * _NS

    G = -(-E // (_CH * NW))
    G = ((G + 3) // 4) * 4
    E_pad = G * NW * _CH
    if E_pad > E:
        npad_rows = jnp.arange(E_pad - E, dtype=jnp.int32)
        pad = jnp.stack(
            [npad_rows % jnp.int32(min(N, _CH)),
             N + npad_rows % jnp.int32(NPAD - N)], axis=0)
        edge_index = jnp.concatenate([edge_index, pad], axis=1)
    edge2 = edge_index
    mesh = plsc.VectorSubcoreMesh(core_axis_name="c", subcore_axis_name="s")

    @functools.partial(
        pl.kernel,
        out_type=jax.ShapeDtypeStruct((_NC, NPAD, D), jnp.float32),
        mesh=mesh,
        scratch_types=[
            pltpu.VMEM((2, 2, _CH), jnp.int32),
            pltpu.VMEM((_CH, D), jnp.float32),
            pltpu.VMEM((_CH, D), jnp.float32),
            pltpu.VMEM_SHARED((NPAD, D), jnp.float32),
            [pltpu.SemaphoreType.DMA] * 2,
        ],
    )
    def agg_kernel(feat_hbm, edge_hbm, zrow_hbm, out_hbm,
                   idx, rows0, rows1, acc, gsems):
        c = lax.axis_index("c")
        s = lax.axis_index("s")
        w = c * _NS + s
        c0 = w * G
        rows = (rows0, rows1)

        def idx_load(k, slot):
            off = (c0 + k) * _CH
            pltpu.sync_copy(edge_hbm.at[0, pl.ds(off, _CH)], idx.at[slot, 0])
            pltpu.sync_copy(edge_hbm.at[1, pl.ds(off, _CH)], idx.at[slot, 1])

        pltpu.sync_copy(zrow_hbm, rows0)

        @pl.loop(0, rows_per_sub // _CH)
        def _(j):
            pltpu.sync_copy(
                rows0, acc.at[pl.ds(s * rows_per_sub + j * _CH, _CH), :])

        plsc.subcore_barrier()

        idx_load(0, 0)
        pltpu.async_copy(feat_hbm.at[idx.at[0, 0]], rows0, gsems[0])

        @pl.loop(0, G - 2, step=2)
        def _(j):
            for b in range(2):
                k = j + b
                nb = 1 - b
                idx_load(k + 1, nb)
                pltpu.async_copy(feat_hbm.at[idx.at[nb, 0]], rows[nb],
                                 gsems[nb])
                pltpu.make_async_copy(feat_hbm.at[idx.at[b, 0]], rows[b],
                                      gsems[b]).wait()
                pltpu.sync_copy(rows[b], acc.at[idx.at[b, 1]], add=True)

        idx_load(G - 1, 1)
        pltpu.async_copy(feat_hbm.at[idx.at[1, 0]], rows1, gsems[1])
        pltpu.make_async_copy(feat_hbm.at[idx.at[0, 0]], rows0,
                              gsems[0]).wait()
        pltpu.sync_copy(rows0, acc.at[idx.at[0, 1]], add=True)
        pltpu.make_async_copy(feat_hbm.at[idx.at[1, 0]], rows1,
                              gsems[1]).wait()
        pltpu.sync_copy(rows1, acc.at[idx.at[1, 1]], add=True)

        plsc.subcore_barrier()

        pltpu.sync_copy(
            acc.at[pl.ds(s * rows_per_sub, rows_per_sub), :],
            out_hbm.at[c, pl.ds(s * rows_per_sub, rows_per_sub), :])

    return agg_kernel(feature, edge2, zrow)


def _tc_fused(feature, partials, W1t, b1, W2t, b2, gamma, beta):
    N, D = feature.shape

    def body(f_ref, p_ref, w1_ref, b1_ref, w2_ref, b2_ref, g_ref, be_ref,
             o_ref):
        x = f_ref[...] + p_ref[0, pl.ds(0, N), :] + p_ref[1, pl.ds(0, N), :]
        h = jnp.dot(x, w1_ref[...], preferred_element_type=jnp.float32,
                    precision=lax.Precision.HIGHEST) + b1_ref[...]
        h = jnp.maximum(h, 0.0)
        h = jnp.dot(h, w2_ref[...], preferred_element_type=jnp.float32,
                    precision=lax.Precision.HIGHEST) + b2_ref[...]
        h = jnp.maximum(h, 0.0)
        mean = jnp.mean(h, axis=0, keepdims=True)
        var = jnp.mean(h * h, axis=0, keepdims=True) - mean * mean
        inv = lax.rsqrt(var + 1e-5)
        o_ref[...] = (h - mean) * inv * g_ref[...] + be_ref[...]

    return pl.pallas_call(
        body,
        out_shape=jax.ShapeDtypeStruct((N, D), jnp.float32),
    )(feature, partials, W1t, b1, W2t, b2, gamma, beta)


def kernel(feature, edge_index, W1, b1, W2, b2, gamma, beta):
    D = feature.shape[1]
    zrow = jnp.zeros((_CH, D), jnp.float32)
    partials = _sc_aggregate(feature, edge_index, zrow)
    return _tc_fused(feature, partials, W1.T, b1.reshape(1, D), W2.T,
                     b2.reshape(1, D), gamma.reshape(1, D),
                     beta.reshape(1, D))


# speedup vs baseline: 3.2261x; 1.2205x over previous
import functools

import jax
import jax.numpy as jnp
from jax import lax
from jax.experimental import pallas as pl
from jax.experimental.pallas import tpu as pltpu
from jax.experimental.pallas import tpu_sc as plsc

_NC = 2
_NS = 16
_CH = 128


def _sc_aggregate(feature, edge_index, zrow):
    N, D = feature.shape
    E = edge_index.shape[1]
    NW = _NC * _NS
    rows_per_sub = ((N + _CH * _NS - 1) // (_CH * _NS)) * _CH
    NPAD = rows_per_sub * _NS

    G = -(-E // (_CH * NW))
    G = ((G + 3) // 4) * 4
    E_pad = G * NW * _CH
    if E_pad > E:
        npad_rows = jnp.arange(E_pad - E, dtype=jnp.int32)
        pad = jnp.stack(
            [npad_rows % jnp.int32(min(N, _CH)),
             N + npad_rows % jnp.int32(NPAD - N)], axis=0)
        edge_index = jnp.concatenate([edge_index, pad], axis=1)
    edge2 = edge_index
    mesh = plsc.VectorSubcoreMesh(core_axis_name="c", subcore_axis_name="s")

    _IB = 16
    NBLK = G // _IB

    @functools.partial(
        pl.kernel,
        out_type=jax.ShapeDtypeStruct((_NC, NPAD, D), jnp.float32),
        mesh=mesh,
        scratch_types=[
            pltpu.VMEM((2, 2, _IB * _CH), jnp.int32),
            pltpu.VMEM((_CH, D), jnp.float32),
            pltpu.VMEM((_CH, D), jnp.float32),
            pltpu.VMEM_SHARED((NPAD, D), jnp.float32),
            [pltpu.SemaphoreType.DMA] * 2,
            [pltpu.SemaphoreType.DMA] * 2,
        ],
    )
    def agg_kernel(feat_hbm, edge_hbm, zrow_hbm, out_hbm,
                   idx, rows0, rows1, acc, gsems, isems):
        c = lax.axis_index("c")
        s = lax.axis_index("s")
        w = c * _NS + s
        c0 = w * G
        rows = (rows0, rows1)

        def idx_block_copy(bi, slot):
            return pltpu.make_async_copy(
                edge_hbm.at[:, pl.ds((c0 + bi * _IB) * _CH, _IB * _CH)],
                idx.at[slot], isems[slot])

        idx_block_copy(0, 0).start()
        pltpu.sync_copy(zrow_hbm, rows0)

        @pl.loop(0, rows_per_sub // _CH)
        def _(j):
            pltpu.sync_copy(
                rows0, acc.at[pl.ds(s * rows_per_sub + j * _CH, _CH), :])

        plsc.subcore_barrier()

        for bi in range(NBLK):
            sl = bi % 2
            idx_block_copy(bi, sl).wait()
            if bi + 1 < NBLK:
                idx_block_copy(bi + 1, 1 - sl).start()

            def launch(u, b):
                pltpu.async_copy(
                    feat_hbm.at[idx.at[sl, 0, pl.ds(u * _CH, _CH)]],
                    rows[b], gsems[b])

            def finish(u, b):
                pltpu.make_async_copy(
                    feat_hbm.at[idx.at[sl, 0, pl.ds(u * _CH, _CH)]],
                    rows[b], gsems[b]).wait()
                pltpu.sync_copy(
                    rows[b], acc.at[idx.at[sl, 1, pl.ds(u * _CH, _CH)]],
                    add=True)

            launch(0, 0)

            @pl.loop(0, _IB - 2, step=2)
            def _(j):
                for b in range(2):
                    u = j + b
                    launch(u + 1, 1 - b)
                    finish(u, b)

            launch(_IB - 1, 1)
            finish(_IB - 2, 0)
            finish(_IB - 1, 1)

        plsc.subcore_barrier()

        pltpu.sync_copy(
            acc.at[pl.ds(s * rows_per_sub, rows_per_sub), :],
            out_hbm.at[c, pl.ds(s * rows_per_sub, rows_per_sub), :])

    return agg_kernel(feature, edge2, zrow)


def _tc_fused(feature, partials, W1t, b1, W2t, b2, gamma, beta):
    N, D = feature.shape

    def body(f_ref, p_ref, w1_ref, b1_ref, w2_ref, b2_ref, g_ref, be_ref,
             o_ref):
        x = f_ref[...] + p_ref[0, pl.ds(0, N), :] + p_ref[1, pl.ds(0, N), :]
        h = jnp.dot(x, w1_ref[...], preferred_element_type=jnp.float32,
                    precision=lax.Precision.HIGHEST) + b1_ref[...]
        h = jnp.maximum(h, 0.0)
        h = jnp.dot(h, w2_ref[...], preferred_element_type=jnp.float32,
                    precision=lax.Precision.HIGHEST) + b2_ref[...]
        h = jnp.maximum(h, 0.0)
        mean = jnp.mean(h, axis=0, keepdims=True)
        var = jnp.mean(h * h, axis=0, keepdims=True) - mean * mean
        inv = lax.rsqrt(var + 1e-5)
        o_ref[...] = (h - mean) * inv * g_ref[...] + be_ref[...]

    return pl.pallas_call(
        body,
        out_shape=jax.ShapeDtypeStruct((N, D), jnp.float32),
    )(feature, partials, W1t, b1, W2t, b2, gamma, beta)


def kernel(feature, edge_index, W1, b1, W2, b2, gamma, beta):
    D = feature.shape[1]
    zrow = jnp.zeros((_CH, D), jnp.float32)
    partials = _sc_aggregate(feature, edge_index, zrow)
    return _tc_fused(feature, partials, W1.T, b1.reshape(1, D), W2.T,
                     b2.reshape(1, D), gamma.reshape(1, D),
                     beta.reshape(1, D))


# speedup vs baseline: 3.3249x; 1.0306x over previous
import functools

import jax
import jax.numpy as jnp
from jax import lax
from jax.experimental import pallas as pl
from jax.experimental.pallas import tpu as pltpu
from jax.experimental.pallas import tpu_sc as plsc

_NC = 2
_NS = 16
_CH = 128


def _sc_aggregate(feature, edge_index, zrow):
    N, D = feature.shape
    E = edge_index.shape[1]
    NW = _NC * _NS
    rows_per_sub = ((N + _CH * _NS - 1) // (_CH * _NS)) * _CH
    NPAD = rows_per_sub * _NS

    G = -(-E // (_CH * NW))
    G = ((G + 3) // 4) * 4
    E_pad = G * NW * _CH

    _IB = 16
    NBLK = G // _IB
    BLK_E = _IB * _CH
    NREALB = E // BLK_E
    npad_rows = jnp.arange(E_pad - E, dtype=jnp.int32)
    pad_edges = jnp.concatenate(
        [edge_index[:, NREALB * BLK_E:],
         jnp.stack([npad_rows % jnp.int32(min(N, _CH)),
                    N + npad_rows % jnp.int32(NPAD - N)], axis=0)], axis=1)
    mesh = plsc.VectorSubcoreMesh(core_axis_name="c", subcore_axis_name="s")

    @functools.partial(
        pl.kernel,
        out_type=jax.ShapeDtypeStruct((_NC, NPAD, D), jnp.float32),
        mesh=mesh,
        scratch_types=[
            pltpu.VMEM((2, 2, _IB * _CH), jnp.int32),
            pltpu.VMEM((_CH, D), jnp.float32),
            pltpu.VMEM((_CH, D), jnp.float32),
            pltpu.VMEM_SHARED((NPAD, D), jnp.float32),
            [pltpu.SemaphoreType.DMA] * 2,
            [pltpu.SemaphoreType.DMA] * 2,
        ],
    )
    def agg_kernel(feat_hbm, edge_hbm, pad_hbm, zrow_hbm, out_hbm,
                   idx, rows0, rows1, acc, gsems, isems):
        c = lax.axis_index("c")
        s = lax.axis_index("s")
        w = c * _NS + s
        rows = (rows0, rows1)

        def idx_block_real(bi, slot):
            gb = w * NBLK + bi
            off = jnp.minimum(gb, NREALB - 1) * BLK_E
            return pltpu.make_async_copy(
                edge_hbm.at[:, pl.ds(off, BLK_E)], idx.at[slot],
                isems[slot])

        def idx_block_start(bi, slot):
            gb = w * NBLK + bi

            @pl.when(gb < NREALB)
            def _():
                idx_block_real(bi, slot).start()

            @pl.when(gb >= NREALB)
            def _():
                off = jnp.maximum(gb - NREALB, 0) * BLK_E
                pltpu.async_copy(
                    pad_hbm.at[:, pl.ds(off, BLK_E)], idx.at[slot],
                    isems[slot])

        idx_block_start(0, 0)
        pltpu.sync_copy(zrow_hbm, rows0)

        @pl.loop(0, rows_per_sub // _CH)
        def _(j):
            pltpu.sync_copy(
                rows0, acc.at[pl.ds(s * rows_per_sub + j * _CH, _CH), :])

        plsc.subcore_barrier()

        for bi in range(NBLK):
            sl = bi % 2
            idx_block_real(bi, sl).wait()
            if bi + 1 < NBLK:
                idx_block_start(bi + 1, 1 - sl)

            def launch(u, b):
                pltpu.async_copy(
                    feat_hbm.at[idx.at[sl, 0, pl.ds(u * _CH, _CH)]],
                    rows[b], gsems[b])

            def finish(u, b):
                pltpu.make_async_copy(
                    feat_hbm.at[idx.at[sl, 0, pl.ds(u * _CH, _CH)]],
                    rows[b], gsems[b]).wait()
                pltpu.sync_copy(
                    rows[b], acc.at[idx.at[sl, 1, pl.ds(u * _CH, _CH)]],
                    add=True)

            launch(0, 0)

            @pl.loop(0, _IB - 2, step=2)
            def _(j):
                for b in range(2):
                    u = j + b
                    launch(u + 1, 1 - b)
                    finish(u, b)

            launch(_IB - 1, 1)
            finish(_IB - 2, 0)
            finish(_IB - 1, 1)

        plsc.subcore_barrier()

        pltpu.sync_copy(
            acc.at[pl.ds(s * rows_per_sub, rows_per_sub), :],
            out_hbm.at[c, pl.ds(s * rows_per_sub, rows_per_sub), :])

    return agg_kernel(feature, edge_index, pad_edges, zrow)


def _tc_fused(feature, partials, W1t, b1, W2t, b2, gamma, beta):
    N, D = feature.shape

    dn = (((1,), (1,)), ((), ()))

    def body(f_ref, p_ref, w1_ref, b1_ref, w2_ref, b2_ref, g_ref, be_ref,
             o_ref):
        x = f_ref[...] + p_ref[0, pl.ds(0, N), :] + p_ref[1, pl.ds(0, N), :]
        h = lax.dot_general(x, w1_ref[...], dn,
                            preferred_element_type=jnp.float32,
                            precision=lax.Precision.HIGHEST) + b1_ref[...]
        h = jnp.maximum(h, 0.0)
        h = lax.dot_general(h, w2_ref[...], dn,
                            preferred_element_type=jnp.float32,
                            precision=lax.Precision.HIGHEST) + b2_ref[...]
        h = jnp.maximum(h, 0.0)
        mean = jnp.mean(h, axis=0, keepdims=True)
        var = jnp.mean(h * h, axis=0, keepdims=True) - mean * mean
        inv = lax.rsqrt(var + 1e-5)
        o_ref[...] = (h - mean) * inv * g_ref[...] + be_ref[...]

    return pl.pallas_call(
        body,
        out_shape=jax.ShapeDtypeStruct((N, D), jnp.float32),
    )(feature, partials, W1t, b1, W2t, b2, gamma, beta)


def kernel(feature, edge_index, W1, b1, W2, b2, gamma, beta):
    D = feature.shape[1]
    zrow = jnp.zeros((_CH, D), jnp.float32)
    partials = _sc_aggregate(feature, edge_index, zrow)
    return _tc_fused(feature, partials, W1, b1.reshape(1, D), W2,
                     b2.reshape(1, D), gamma.reshape(1, D),
                     beta.reshape(1, D))


# speedup vs baseline: 3.4513x; 1.0380x over previous
import functools

import jax
import jax.numpy as jnp
from jax import lax
from jax.experimental import pallas as pl
from jax.experimental.pallas import tpu as pltpu
from jax.experimental.pallas import tpu_sc as plsc

_NC = 2
_NS = 16
_CH = 128


def _sc_aggregate(feature, edge_index, zrow):
    N, D = feature.shape
    E = edge_index.shape[1]
    NW = _NC * _NS
    rows_per_sub = ((N + _CH * _NS - 1) // (_CH * _NS)) * _CH
    NPAD = rows_per_sub * _NS

    G = -(-E // (_CH * NW))
    G = ((G + 3) // 4) * 4
    E_pad = G * NW * _CH

    _IB = 16
    NBLK = G // _IB
    BLK_E = _IB * _CH
    NREALB = E // BLK_E
    npad_rows = jnp.arange(E_pad - E, dtype=jnp.int32)
    pad_edges = jnp.concatenate(
        [edge_index[:, NREALB * BLK_E:],
         jnp.stack([npad_rows % jnp.int32(min(N, _CH)),
                    N + npad_rows % jnp.int32(NPAD - N)], axis=0)], axis=1)
    mesh = plsc.VectorSubcoreMesh(core_axis_name="c", subcore_axis_name="s")

    @functools.partial(
        pl.kernel,
        out_type=jax.ShapeDtypeStruct((_NC, NPAD, D), jnp.float32),
        mesh=mesh,
        scratch_types=[
            pltpu.VMEM((2, 2, _IB * _CH), jnp.int32),
            pltpu.VMEM((_CH, D), jnp.float32),
            pltpu.VMEM((_CH, D), jnp.float32),
            pltpu.VMEM_SHARED((NPAD, D), jnp.float32),
            [pltpu.SemaphoreType.DMA] * 2,
            [pltpu.SemaphoreType.DMA] * 2,
        ],
    )
    def agg_kernel(feat_hbm, edge_hbm, pad_hbm, zrow_hbm, out_hbm,
                   idx, rows0, rows1, acc, gsems, isems):
        c = lax.axis_index("c")
        s = lax.axis_index("s")
        w = c * _NS + s
        rows = (rows0, rows1)

        def idx_block_real(bi, slot):
            gb = w * NBLK + bi
            off = jnp.minimum(gb, NREALB - 1) * BLK_E
            return pltpu.make_async_copy(
                edge_hbm.at[:, pl.ds(off, BLK_E)], idx.at[slot],
                isems[slot])

        def idx_block_start(bi, slot):
            gb = w * NBLK + bi

            @pl.when(gb < NREALB)
            def _():
                idx_block_real(bi, slot).start()

            @pl.when(gb >= NREALB)
            def _():
                off = jnp.maximum(gb - NREALB, 0) * BLK_E
                pltpu.async_copy(
                    pad_hbm.at[:, pl.ds(off, BLK_E)], idx.at[slot],
                    isems[slot])

        idx_block_start(0, 0)
        pltpu.sync_copy(zrow_hbm, rows0)

        @pl.loop(0, rows_per_sub // _CH)
        def _(j):
            pltpu.sync_copy(
                rows0, acc.at[pl.ds(s * rows_per_sub + j * _CH, _CH), :])

        plsc.subcore_barrier()

        def launch(sl, u, b):
            pltpu.async_copy(
                feat_hbm.at[idx.at[sl, 0, pl.ds(u * _CH, _CH)]],
                rows[b], gsems[b])

        def finish(sl, u, b):
            pltpu.make_async_copy(
                feat_hbm.at[idx.at[sl, 0, pl.ds(u * _CH, _CH)]],
                rows[b], gsems[b]).wait()
            pltpu.sync_copy(
                rows[b], acc.at[idx.at[sl, 1, pl.ds(u * _CH, _CH)]],
                add=True)

        def steady(sl):
            @pl.loop(0, _IB - 2, step=2)
            def _(j):
                for b in range(2):
                    u = j + b
                    finish(sl, u, b)
                    launch(sl, u + 2, b)

        idx_block_real(0, 0).wait()
        launch(0, 0, 0)
        launch(0, 1, 1)
        for bi in range(NBLK):
            sl = bi % 2
            if bi + 1 < NBLK:
                idx_block_start(bi + 1, 1 - sl)
                steady(sl)
                idx_block_real(bi + 1, 1 - sl).wait()
                finish(sl, _IB - 2, 0)
                launch(1 - sl, 0, 0)
                finish(sl, _IB - 1, 1)
                launch(1 - sl, 1, 1)
            else:
                steady(sl)
                finish(sl, _IB - 2, 0)
                finish(sl, _IB - 1, 1)

        plsc.subcore_barrier()

        pltpu.sync_copy(
            acc.at[pl.ds(s * rows_per_sub, rows_per_sub), :],
            out_hbm.at[c, pl.ds(s * rows_per_sub, rows_per_sub), :])

    return agg_kernel(feature, edge_index, pad_edges, zrow)


def _tc_fused(feature, partials, W1t, b1, W2t, b2, gamma, beta):
    N, D = feature.shape

    dn = (((1,), (1,)), ((), ()))

    def body(f_ref, p_ref, w1_ref, b1_ref, w2_ref, b2_ref, g_ref, be_ref,
             o_ref):
        x = f_ref[...] + p_ref[0, pl.ds(0, N), :] + p_ref[1, pl.ds(0, N), :]
        h = lax.dot_general(x, w1_ref[...], dn,
                            preferred_element_type=jnp.float32,
                            precision=lax.Precision.HIGHEST) + b1_ref[...]
        h = jnp.maximum(h, 0.0)
        h = lax.dot_general(h, w2_ref[...], dn,
                            preferred_element_type=jnp.float32,
                            precision=lax.Precision.HIGHEST) + b2_ref[...]
        h = jnp.maximum(h, 0.0)
        mean = jnp.mean(h, axis=0, keepdims=True)
        var = jnp.mean(h * h, axis=0, keepdims=True) - mean * mean
        inv = lax.rsqrt(var + 1e-5)
        o_ref[...] = (h - mean) * inv * g_ref[...] + be_ref[...]

    return pl.pallas_call(
        body,
        out_shape=jax.ShapeDtypeStruct((N, D), jnp.float32),
    )(feature, partials, W1t, b1, W2t, b2, gamma, beta)


def kernel(feature, edge_index, W1, b1, W2, b2, gamma, beta):
    D = feature.shape[1]
    zrow = jnp.zeros((_CH, D), jnp.float32)
    partials = _sc_aggregate(feature, edge_index, zrow)
    return _tc_fused(feature, partials, W1, b1.reshape(1, D), W2,
                     b2.reshape(1, D), gamma.reshape(1, D),
                     beta.reshape(1, D))
